# baseline (speedup 1.0000x reference)
import functools

import jax
import jax.numpy as jnp
from jax import lax
from jax.experimental import pallas as pl
from jax.experimental.pallas import tpu as pltpu
from jax.experimental.pallas import tpu_sc as plsc
from jax.experimental.layout import Format, Layout

NUM_CORES = 2
NUM_SUBCORES = 16
NW = NUM_CORES * NUM_SUBCORES

CHUNK = 128
LANE = 16


@functools.partial(jax.jit, static_argnames=("v", "dp"))
def _gather_rows(idx, table_pad, *, v, dp):
    nw, n_chunks, _ = idx.shape
    n = nw * n_chunks * CHUNK
    mesh = plsc.VectorSubcoreMesh(core_axis_name="c", subcore_axis_name="s")

    @functools.partial(
        pl.kernel,
        out_type=jax.ShapeDtypeStruct((n, dp), jnp.float32),
        mesh=mesh,
        scratch_types=[
            pltpu.VMEM((n_chunks, CHUNK), jnp.int32),
            pltpu.VMEM((2, CHUNK, dp), jnp.float32),
            pltpu.SemaphoreType.DMA((2,)),
            pltpu.SemaphoreType.DMA((2,)),
        ],
        compiler_params=pltpu.CompilerParams(use_tc_tiling_on_sc=False),
    )
    def run(idx_hbm, table_hbm, out_hbm, idx_v, bufs, gsem, wsem):
        wid = lax.axis_index("s") * NUM_CORES + lax.axis_index("c")
        base = wid * n_chunks * CHUNK
        pltpu.sync_copy(idx_hbm.at[wid], idx_v)

        def gather(i, b):
            return pltpu.make_async_copy(
                table_hbm.at[idx_v.at[i]], bufs.at[b], gsem.at[b]
            )

        def write(i, b):
            return pltpu.make_async_copy(
                bufs.at[b], out_hbm.at[pl.ds(base + i * CHUNK, CHUNK)], wsem.at[b]
            )

        for b in range(2):
            gather(b, b).start()

        def body(g, carry):
            for b in range(2):
                i = g * 2 + b
                gather(i, b).wait()
                write(i, b).start()
                nxt = i + 2

                @pl.when(nxt < n_chunks)
                def _():
                    write(i, b).wait()
                    gather(nxt, b).start()

            return carry

        lax.fori_loop(0, n_chunks // 2, body, 0)
        for b in range(2):
            write(0, b).wait()

    return run(idx, table_pad)


def kernel(x, table):
    b, l = x.shape
    v, d = table.shape
    n = b * l
    dp = (d + LANE - 1) // LANE * LANE
    idx = x.reshape(NW, n // (NW * CHUNK), CHUNK).astype(jnp.int32)
    fmt = Format(
        Layout(major_to_minor=(0, 1), tiling=((16,),)),
        jax.sharding.SingleDeviceSharding(jax.devices()[0]),
    )
    table_pad = jax.device_put(jnp.pad(table, ((0, 0), (0, dp - d))), fmt)
    out = _gather_rows(idx, table_pad, v=v, dp=dp)
    return out.reshape(b, l, dp)[:, None, :, :d]


# speedup vs baseline: 1.0003x; 1.0003x over previous
import functools

import jax
import jax.numpy as jnp
from jax import lax
from jax.experimental import pallas as pl
from jax.experimental.pallas import tpu as pltpu
from jax.experimental.pallas import tpu_sc as plsc

NUM_CORES = 2
NUM_SUBCORES = 16
NW = NUM_CORES * NUM_SUBCORES

CHUNK = 128
LANE = 16


@functools.partial(jax.jit, static_argnames=("v", "dp"))
def _gather_rows(idx, table_pad, *, v, dp):
    nw, n_chunks, _ = idx.shape
    n = nw * n_chunks * CHUNK
    mesh = plsc.VectorSubcoreMesh(core_axis_name="c", subcore_axis_name="s")

    @functools.partial(
        pl.kernel,
        out_type=jax.ShapeDtypeStruct((n, dp), jnp.float32),
        mesh=mesh,
        scratch_types=[
            pltpu.VMEM((n_chunks, CHUNK), jnp.int32),
            pltpu.VMEM((2, CHUNK, dp), jnp.float32),
            pltpu.SemaphoreType.DMA((2,)),
            pltpu.SemaphoreType.DMA((2,)),
        ],
        compiler_params=pltpu.CompilerParams(use_tc_tiling_on_sc=False),
    )
    def run(idx_hbm, table_hbm, out_hbm, idx_v, bufs, gsem, wsem):
        wid = lax.axis_index("s") * NUM_CORES + lax.axis_index("c")
        base = wid * n_chunks * CHUNK
        pltpu.sync_copy(idx_hbm.at[wid], idx_v)

        def gather(i, b):
            return pltpu.make_async_copy(
                table_hbm.at[idx_v.at[i]], bufs.at[b], gsem.at[b]
            )

        def write(i, b):
            return pltpu.make_async_copy(
                bufs.at[b], out_hbm.at[pl.ds(base + i * CHUNK, CHUNK)], wsem.at[b]
            )

        for b in range(2):
            gather(b, b).start()

        def body(g, carry):
            for b in range(2):
                i = g * 2 + b
                gather(i, b).wait()
                write(i, b).start()
                nxt = i + 2

                @pl.when(nxt < n_chunks)
                def _():
                    write(i, b).wait()
                    gather(nxt, b).start()

            return carry

        lax.fori_loop(0, n_chunks // 2, body, 0)
        for b in range(2):
            write(0, b).wait()

    return run(idx, table_pad)


def kernel(x, table):
    b, l = x.shape
    v, d = table.shape
    n = b * l
    dp = (d + LANE - 1) // LANE * LANE
    idx = x.reshape(NW, n // (NW * CHUNK), CHUNK).astype(jnp.int32)
    table_pad = jnp.pad(table, ((0, 0), (0, dp - d)))
    out = _gather_rows(idx, table_pad, v=v, dp=dp)
    return out.reshape(b, l, dp)[:, None, :, :d]
